# baseline (speedup 1.0000x reference)
import jax
import jax.numpy as jnp
from jax import lax
from jax.experimental import pallas as pl
from jax.experimental.pallas import tpu as pltpu
from jax.experimental.pallas import tpu_sc as plsc

EMB = 64
BATCH = 4096
HIST = 50

B = BATCH * HIST
NW = 32
BPW = B // NW
GROW = 128
NG = BPW // GROW
CHUNK_G = 5
RPC = CHUNK_G * GROW
NCHUNK = NG // CHUNK_G
GPC = RPC // 16


def _sc_body(table_hbm, idx_hbm, out_hbm, idx_v, rows_v, sem):
    wid = lax.axis_index("s") * 2 + lax.axis_index("c")
    base = wid * BPW
    pltpu.sync_copy(idx_hbm.at[wid], idx_v)

    def chunk_body(i, carry):
        descs = [
            pltpu.async_copy(
                table_hbm.at[idx_v.at[i * CHUNK_G + j]],
                rows_v.at[pl.ds(j * GROW, GROW)],
                sem,
            )
            for j in range(CHUNK_G)
        ]
        for d in descs:
            d.wait()

        def grp_body(g, c2):
            jj = i * CHUNK_G + g // (GROW // 16)
            off = (g % (GROW // 16)) * 16
            iv = idx_v[jj, pl.ds(off, 16)]
            msk = iv == 0
            rowvec = g * 16 + lax.iota(jnp.int32, 16)
            z = jnp.zeros((16,), jnp.float32)
            for c in range(EMB):
                colvec = jnp.full((16,), c, jnp.int32)
                plsc.store_scatter(rows_v, [rowvec, colvec], z, mask=msk)
            return c2

        lax.fori_loop(0, GPC, grp_body, 0)
        pltpu.sync_copy(rows_v, out_hbm.at[pl.ds(base + i * RPC, RPC)])
        return carry

    lax.fori_loop(0, NCHUNK, chunk_body, 0)


def kernel(inputs, table):
    idx = inputs.reshape(-1).astype(jnp.int32).reshape(NW, NG, GROW)
    mesh = plsc.VectorSubcoreMesh(core_axis_name="c", subcore_axis_name="s")
    k = pl.kernel(
        _sc_body,
        out_type=jax.ShapeDtypeStruct((B, EMB), jnp.float32),
        mesh=mesh,
        compiler_params=pltpu.CompilerParams(
            use_tc_tiling_on_sc=False, needs_layout_passes=False
        ),
        scratch_types=[
            pltpu.VMEM((NG, GROW), jnp.int32),
            pltpu.VMEM((RPC, EMB), jnp.float32),
            pltpu.SemaphoreType.DMA,
        ],
    )
    out = k(table, idx)
    return out.reshape(BATCH, HIST, EMB)


# speedup vs baseline: 1.0184x; 1.0184x over previous
import jax
import jax.numpy as jnp
from jax import lax
from jax.experimental import pallas as pl
from jax.experimental.pallas import tpu as pltpu
from jax.experimental.pallas import tpu_sc as plsc

EMB = 64
BATCH = 4096
HIST = 50

NW = 32
BPW = BATCH // NW
NB = 16
NCHUNK = BPW // NB
ROWS_PC = NB * HIST
GPC = ROWS_PC // 16


def _fixup(idx_v, rows_v, b0):
    def grp_body(g, carry):
        rowvec = g * 16 + lax.iota(jnp.int32, 16)
        bvec = rowvec // HIST
        hvec = rowvec % HIST
        iv = plsc.load_gather(idx_v, [b0 + bvec, hvec])
        msk = iv == 0
        z = jnp.zeros((16,), jnp.float32)
        for c in range(EMB):
            colvec = jnp.full((16,), c, jnp.int32)
            plsc.store_scatter(rows_v, [bvec, hvec, colvec], z, mask=msk)
        return carry

    lax.fori_loop(0, GPC, grp_body, 0)


def _sc_body(
    table_hbm, idx_hbm, out_hbm, idx_v, rows_a, rows_b, gsem_a, gsem_b, wsem_a, wsem_b
):
    wid = lax.axis_index("s") * 2 + lax.axis_index("c")
    base = wid * BPW
    pltpu.sync_copy(idx_hbm.at[pl.ds(base, BPW)], idx_v)

    bufs = (rows_a, rows_b)
    gsems = (gsem_a, gsem_b)
    wsems = (wsem_a, wsem_b)

    def fire_gathers(i):
        buf, sem = bufs[i % 2], gsems[i % 2]
        return [
            pltpu.async_copy(table_hbm.at[idx_v.at[i * NB + jb]], buf.at[jb], sem)
            for jb in range(NB)
        ]

    flush_desc = [None, None]
    g_descs = fire_gathers(0)
    for i in range(NCHUNK):
        b = i % 2
        nxt = None
        if i + 1 < NCHUNK:
            nb = (i + 1) % 2
            if flush_desc[nb] is not None:
                flush_desc[nb].wait()
                flush_desc[nb] = None
            nxt = fire_gathers(i + 1)
        for d in g_descs:
            d.wait()
        _fixup(idx_v, bufs[b], i * NB)
        flush_desc[b] = pltpu.async_copy(
            bufs[b], out_hbm.at[pl.ds(base + i * NB, NB)], wsems[b]
        )
        g_descs = nxt
    for fd in flush_desc:
        if fd is not None:
            fd.wait()


def kernel(inputs, table):
    idx = inputs.astype(jnp.int32)
    mesh = plsc.VectorSubcoreMesh(core_axis_name="c", subcore_axis_name="s")
    k = pl.kernel(
        _sc_body,
        out_type=jax.ShapeDtypeStruct((BATCH, HIST, EMB), jnp.float32),
        mesh=mesh,
        scratch_types=[
            pltpu.VMEM((BPW, HIST), jnp.int32),
            pltpu.VMEM((NB, HIST, EMB), jnp.float32),
            pltpu.VMEM((NB, HIST, EMB), jnp.float32),
            pltpu.SemaphoreType.DMA,
            pltpu.SemaphoreType.DMA,
            pltpu.SemaphoreType.DMA,
            pltpu.SemaphoreType.DMA,
        ],
        compiler_params=pltpu.CompilerParams(
            use_tc_tiling_on_sc=False, needs_layout_passes=False
        ),
    )
    return k(table, idx)


# speedup vs baseline: 1.1791x; 1.1578x over previous
import jax
import jax.numpy as jnp
from jax import lax
from jax.experimental import pallas as pl
from jax.experimental.pallas import tpu as pltpu
from jax.experimental.pallas import tpu_sc as plsc

EMB = 64
EMBP = 128
BATCH = 4096
HIST = 50
ZROW = 1000000

NW = 32
BPW = BATCH // NW
NB = 8
NCHUNK = BPW // NB


def _sc_body(
    table_hbm, idx_hbm, out_hbm,
    idx_a, idx_b, rows_a, rows_b, gsem_a, gsem_b, wsem_a, wsem_b,
):
    wid = lax.axis_index("s") * 2 + lax.axis_index("c")
    base = wid * BPW

    idxs = (idx_a, idx_b)
    bufs = (rows_a, rows_b)
    gsems = (gsem_a, gsem_b)
    wsems = (wsem_a, wsem_b)

    def stage_idx(i):
        pltpu.sync_copy(idx_hbm.at[pl.ds(base + i * NB, NB)], idxs[i % 2])

    def fire_gathers(i):
        iv, buf, sem = idxs[i % 2], bufs[i % 2], gsems[i % 2]
        return [
            pltpu.async_copy(table_hbm.at[iv.at[jb]], buf.at[jb], sem)
            for jb in range(NB)
        ]

    flush_desc = [None, None]
    stage_idx(0)
    g_descs = fire_gathers(0)
    for i in range(NCHUNK):
        b = i % 2
        nxt = None
        if i + 1 < NCHUNK:
            nb = (i + 1) % 2
            if flush_desc[nb] is not None:
                flush_desc[nb].wait()
                flush_desc[nb] = None
            stage_idx(i + 1)
            nxt = fire_gathers(i + 1)
        for d in g_descs:
            d.wait()
        flush_desc[b] = pltpu.async_copy(
            bufs[b], out_hbm.at[pl.ds(base + i * NB, NB)], wsems[b]
        )
        g_descs = nxt
    for fd in flush_desc:
        if fd is not None:
            fd.wait()


def kernel(inputs, table):
    idx = inputs.astype(jnp.int32)
    idx = jnp.where(idx == 0, ZROW, idx)
    tpad = jnp.pad(table, ((0, 8), (0, EMBP - EMB)))
    mesh = plsc.VectorSubcoreMesh(core_axis_name="c", subcore_axis_name="s")
    k = pl.kernel(
        _sc_body,
        out_type=jax.ShapeDtypeStruct((BATCH, HIST, EMBP), jnp.float32),
        mesh=mesh,
        scratch_types=[
            pltpu.VMEM((NB, HIST), jnp.int32),
            pltpu.VMEM((NB, HIST), jnp.int32),
            pltpu.VMEM((NB, HIST, EMBP), jnp.float32),
            pltpu.VMEM((NB, HIST, EMBP), jnp.float32),
            pltpu.SemaphoreType.DMA,
            pltpu.SemaphoreType.DMA,
            pltpu.SemaphoreType.DMA,
            pltpu.SemaphoreType.DMA,
        ],
        compiler_params=pltpu.CompilerParams(
            use_tc_tiling_on_sc=True, needs_layout_passes=False
        ),
    )
    out = k(tpad, idx)
    return out[:, :, :EMB]
